# baseline (speedup 1.0000x reference)
import jax
import jax.numpy as jnp
from jax.experimental import pallas as pl
from jax.experimental.pallas import tpu as pltpu

_NCHUNKS = 32


def _dma_copy_kernel(in_ref, out_ref, vbuf, in_sem, out_sem):
    rows = in_ref.shape[0]
    bm = rows // _NCHUNKS
    loads = [
        pltpu.make_async_copy(
            in_ref.at[pl.ds(k * bm, bm)], vbuf.at[pl.ds(k * bm, bm)], in_sem.at[k]
        )
        for k in range(_NCHUNKS)
    ]
    stores = [
        pltpu.make_async_copy(
            vbuf.at[pl.ds(k * bm, bm)], out_ref.at[pl.ds(k * bm, bm)], out_sem.at[k]
        )
        for k in range(_NCHUNKS)
    ]
    for k in range(_NCHUNKS):
        loads[k].start()
    for k in range(_NCHUNKS):
        loads[k].wait()
        stores[k].start()
    for k in range(_NCHUNKS):
        stores[k].wait()


def kernel(seq_len, position_embedding):
    del seq_len
    M, H = position_embedding.shape
    return pl.pallas_call(
        _dma_copy_kernel,
        in_specs=[pl.BlockSpec(memory_space=pltpu.MemorySpace.HBM)],
        out_specs=pl.BlockSpec(memory_space=pltpu.MemorySpace.HBM),
        out_shape=jax.ShapeDtypeStruct((M, H), position_embedding.dtype),
        scratch_shapes=[
            pltpu.VMEM((M, H), position_embedding.dtype),
            pltpu.SemaphoreType.DMA((_NCHUNKS,)),
            pltpu.SemaphoreType.DMA((_NCHUNKS,)),
        ],
    )(position_embedding)


# speedup vs baseline: 1.0321x; 1.0321x over previous
import jax
import jax.numpy as jnp
from jax.experimental import pallas as pl
from jax.experimental.pallas import tpu as pltpu

_NCHUNKS = 8


def _dma_copy_kernel(in_ref, out_ref, vbuf, in_sem, out_sem):
    rows = in_ref.shape[0]
    bm = rows // _NCHUNKS
    loads = [
        pltpu.make_async_copy(
            in_ref.at[pl.ds(k * bm, bm)], vbuf.at[pl.ds(k * bm, bm)], in_sem.at[k]
        )
        for k in range(_NCHUNKS)
    ]
    stores = [
        pltpu.make_async_copy(
            vbuf.at[pl.ds(k * bm, bm)], out_ref.at[pl.ds(k * bm, bm)], out_sem.at[k]
        )
        for k in range(_NCHUNKS)
    ]
    for k in range(_NCHUNKS):
        loads[k].start()
    for k in range(_NCHUNKS):
        loads[k].wait()
        stores[k].start()
    for k in range(_NCHUNKS):
        stores[k].wait()


def kernel(seq_len, position_embedding):
    del seq_len
    M, H = position_embedding.shape
    return pl.pallas_call(
        _dma_copy_kernel,
        in_specs=[pl.BlockSpec(memory_space=pltpu.MemorySpace.HBM)],
        out_specs=pl.BlockSpec(memory_space=pltpu.MemorySpace.HBM),
        out_shape=jax.ShapeDtypeStruct((M, H), position_embedding.dtype),
        scratch_shapes=[
            pltpu.VMEM((M, H), position_embedding.dtype),
            pltpu.SemaphoreType.DMA((_NCHUNKS,)),
            pltpu.SemaphoreType.DMA((_NCHUNKS,)),
        ],
    )(position_embedding)


# speedup vs baseline: 1.0373x; 1.0051x over previous
import jax
import jax.numpy as jnp
from jax.experimental import pallas as pl
from jax.experimental.pallas import tpu as pltpu

_NCHUNKS = 4


def _dma_copy_kernel(in_ref, out_ref, vbuf, in_sem, out_sem):
    rows = in_ref.shape[0]
    bm = rows // _NCHUNKS
    loads = [
        pltpu.make_async_copy(
            in_ref.at[pl.ds(k * bm, bm)], vbuf.at[pl.ds(k * bm, bm)], in_sem.at[k]
        )
        for k in range(_NCHUNKS)
    ]
    stores = [
        pltpu.make_async_copy(
            vbuf.at[pl.ds(k * bm, bm)], out_ref.at[pl.ds(k * bm, bm)], out_sem.at[k]
        )
        for k in range(_NCHUNKS)
    ]
    for k in range(_NCHUNKS):
        loads[k].start()
    for k in range(_NCHUNKS):
        loads[k].wait()
        stores[k].start()
    for k in range(_NCHUNKS):
        stores[k].wait()


def kernel(seq_len, position_embedding):
    del seq_len
    M, H = position_embedding.shape
    return pl.pallas_call(
        _dma_copy_kernel,
        in_specs=[pl.BlockSpec(memory_space=pltpu.MemorySpace.HBM)],
        out_specs=pl.BlockSpec(memory_space=pltpu.MemorySpace.HBM),
        out_shape=jax.ShapeDtypeStruct((M, H), position_embedding.dtype),
        scratch_shapes=[
            pltpu.VMEM((M, H), position_embedding.dtype),
            pltpu.SemaphoreType.DMA((_NCHUNKS,)),
            pltpu.SemaphoreType.DMA((_NCHUNKS,)),
        ],
    )(position_embedding)


# speedup vs baseline: 1.0568x; 1.0188x over previous
import jax
import jax.numpy as jnp
from jax.experimental import pallas as pl
from jax.experimental.pallas import tpu as pltpu

_NCHUNKS = 2


def _dma_copy_kernel(in_ref, out_ref, vbuf, in_sem, out_sem):
    rows = in_ref.shape[0]
    bm = rows // _NCHUNKS
    loads = [
        pltpu.make_async_copy(
            in_ref.at[pl.ds(k * bm, bm)], vbuf.at[pl.ds(k * bm, bm)], in_sem.at[k]
        )
        for k in range(_NCHUNKS)
    ]
    stores = [
        pltpu.make_async_copy(
            vbuf.at[pl.ds(k * bm, bm)], out_ref.at[pl.ds(k * bm, bm)], out_sem.at[k]
        )
        for k in range(_NCHUNKS)
    ]
    for k in range(_NCHUNKS):
        loads[k].start()
    for k in range(_NCHUNKS):
        loads[k].wait()
        stores[k].start()
    for k in range(_NCHUNKS):
        stores[k].wait()


def kernel(seq_len, position_embedding):
    del seq_len
    M, H = position_embedding.shape
    return pl.pallas_call(
        _dma_copy_kernel,
        in_specs=[pl.BlockSpec(memory_space=pltpu.MemorySpace.HBM)],
        out_specs=pl.BlockSpec(memory_space=pltpu.MemorySpace.HBM),
        out_shape=jax.ShapeDtypeStruct((M, H), position_embedding.dtype),
        scratch_shapes=[
            pltpu.VMEM((M, H), position_embedding.dtype),
            pltpu.SemaphoreType.DMA((_NCHUNKS,)),
            pltpu.SemaphoreType.DMA((_NCHUNKS,)),
        ],
    )(position_embedding)
